# baseline (speedup 1.0000x reference)
import functools

import jax
import jax.numpy as jnp
from jax import lax
from jax.experimental import pallas as pl
from jax.experimental.pallas import tpu as pltpu
from jax.experimental.pallas import tpu_sc as plsc

N_NODES = 10000
N_EDGES = 320000
D_FEAT = 128

NUM_CORES = 2
NUM_SUBCORES = 16
NW = NUM_CORES * NUM_SUBCORES
PER_W = N_EDGES // NW
CHUNK = 128
NFULL = PER_W // CHUNK
TAIL = PER_W - NFULL * CHUNK
NPAIR = NFULL // 2
FILL = N_NODES // 2
POS_MAX = 2 * N_EDGES - 1


def _gather_kernel(pairs_hbm, px_hbm, out_hbm, table_sp,
                   pos_u, p0, p1, i0, i1, r0, r1,
                   isem0, isem1, gsem0, gsem1):
    sid = lax.axis_index("s")
    wid = sid * NUM_CORES + lax.axis_index("c")
    base = wid * PER_W

    @pl.when(sid < 2)
    def _fill():
        pltpu.sync_copy(px_hbm.at[pl.ds(sid * FILL, FILL)],
                        table_sp.at[pl.ds(sid * FILL, FILL)])

    lane = lax.iota(jnp.int32, 16)
    pos = (p0, p1)
    idx = (i0, i1)
    rows = (r0, r1)
    isems = (isem0, isem1)
    gsems = (gsem0, gsem1)

    for j in range(CHUNK // 16):
        pos_u[pl.ds(16 * j, 16)] = 2 * base + 32 * j + 1 + 2 * lane

    def advance(b):
        for j in range(CHUNK // 16):
            v = pos_u[pl.ds(16 * j, 16)] + 2 * CHUNK
            pos_u[pl.ds(16 * j, 16)] = v
            pos[b][pl.ds(16 * j, 16)] = jnp.minimum(v, POS_MAX)

    def start_idx(b):
        pltpu.async_copy(pairs_hbm.at[pos[b]], idx[b], isems[b])

    def wait_idx(b):
        pltpu.make_async_copy(pairs_hbm.at[pos[b]], idx[b], isems[b]).wait()

    def start_px(b):
        pltpu.async_copy(table_sp.at[idx[b]], rows[b], gsems[b])

    def wait_px(b):
        pltpu.make_async_copy(table_sp.at[idx[b]], rows[b], gsems[b]).wait()

    for j in range(CHUNK // 16):
        p0[pl.ds(16 * j, 16)] = pos_u[pl.ds(16 * j, 16)]
    start_idx(0)
    advance(1)
    start_idx(1)
    plsc.subcore_barrier()
    wait_idx(0)
    start_px(0)

    def body(p, _):
        i = 2 * p
        advance(0)
        wait_px(0)
        start_idx(0)
        wait_idx(1)
        start_px(1)
        pltpu.sync_copy(rows[0], out_hbm.at[pl.ds(base + i * CHUNK, CHUNK)])
        advance(1)
        wait_px(1)
        @pl.when(p < NPAIR - 1)
        def _next():
            start_idx(1)
        wait_idx(0)
        start_px(0)
        pltpu.sync_copy(rows[1],
                        out_hbm.at[pl.ds(base + (i + 1) * CHUNK, CHUNK)])
        return 0

    lax.fori_loop(0, NPAIR, body, 0)

    wait_px(0)
    pltpu.sync_copy(rows[0].at[pl.ds(0, TAIL)],
                    out_hbm.at[pl.ds(base + NFULL * CHUNK, TAIL)])


@jax.jit
def _pix_gather(pairs_flat, px):
    mesh = plsc.VectorSubcoreMesh(core_axis_name="c", subcore_axis_name="s")
    run = functools.partial(
        pl.kernel,
        mesh=mesh,
        out_type=jax.ShapeDtypeStruct((N_EDGES, D_FEAT), jnp.float32),
        scratch_types=[
            pltpu.VMEM_SHARED((N_NODES, D_FEAT), jnp.float32),
            pltpu.VMEM((CHUNK,), jnp.int32),
            pltpu.VMEM((CHUNK,), jnp.int32),
            pltpu.VMEM((CHUNK,), jnp.int32),
            pltpu.VMEM((CHUNK,), jnp.int32),
            pltpu.VMEM((CHUNK,), jnp.int32),
            pltpu.VMEM((CHUNK, D_FEAT), jnp.float32),
            pltpu.VMEM((CHUNK, D_FEAT), jnp.float32),
            pltpu.SemaphoreType.DMA,
            pltpu.SemaphoreType.DMA,
            pltpu.SemaphoreType.DMA,
            pltpu.SemaphoreType.DMA,
        ],
    )(_gather_kernel)
    return run(pairs_flat, px)


def kernel(ind_2, px):
    return _pix_gather(ind_2.reshape(2 * N_EDGES), px)


# speedup vs baseline: 1.0133x; 1.0133x over previous
import functools

import jax
import jax.numpy as jnp
from jax import lax
from jax.experimental import pallas as pl
from jax.experimental.pallas import tpu as pltpu
from jax.experimental.pallas import tpu_sc as plsc

N_NODES = 10000
N_EDGES = 320000
D_FEAT = 128

NUM_CORES = 2
NUM_SUBCORES = 16
NW = NUM_CORES * NUM_SUBCORES
PER_W = N_EDGES // NW
CHUNK = 128
NFULL = PER_W // CHUNK
TAIL = PER_W - NFULL * CHUNK
NPAIR = NFULL // 2
FILL = N_NODES // 2
GROUPS = CHUNK // 16


def _gather_kernel(pairs_hbm, px_hbm, out_hbm, table_sp,
                   pb0, pb1, ib0, ib1, r0, r1,
                   psem0, psem1, gsem0, gsem1):
    sid = lax.axis_index("s")
    wid = sid * NUM_CORES + lax.axis_index("c")
    base = wid * PER_W

    @pl.when(sid < 2)
    def _fill():
        pltpu.sync_copy(px_hbm.at[pl.ds(sid * FILL, FILL)],
                        table_sp.at[pl.ds(sid * FILL, FILL)])

    lane = lax.iota(jnp.int32, 16)
    odd_map = jnp.where(lane < 8, 2 * lane + 1, 2 * lane - 15)
    pbuf = (pb0, pb1)
    ibuf = (ib0, ib1)
    rows = (r0, r1)
    psems = (psem0, psem1)
    gsems = (gsem0, gsem1)

    def start_pairs(i, b):
        pltpu.async_copy(pairs_hbm.at[pl.ds(2 * base + 2 * CHUNK * i,
                                            2 * CHUNK)],
                         pbuf[b], psems[b])

    def wait_pairs(b):
        pltpu.make_async_copy(pairs_hbm.at[pl.ds(0, 2 * CHUNK)],
                              pbuf[b], psems[b]).wait()

    dnums = lax.GatherDimensionNumbers(
        offset_dims=(), collapsed_slice_dims=(0,), start_index_map=(0,))

    def vgather(v, idx16):
        return lax.gather(v, idx16[:, None], dnums, (1,),
                          mode=lax.GatherScatterMode.PROMISE_IN_BOUNDS)

    def deint(b, groups=GROUPS):
        for j in range(groups):
            v0 = pbuf[b][pl.ds(32 * j, 16)]
            v1 = pbuf[b][pl.ds(32 * j + 16, 16)]
            t0 = vgather(v0, odd_map)
            t1 = vgather(v1, odd_map)
            ibuf[b][pl.ds(16 * j, 16)] = jnp.where(lane < 8, t0, t1)

    def start_px(b):
        pltpu.async_copy(table_sp.at[ibuf[b]], rows[b], gsems[b])

    def wait_px(b):
        pltpu.make_async_copy(table_sp.at[ibuf[b]], rows[b], gsems[b]).wait()

    def scatter(i, b):
        pltpu.sync_copy(rows[b], out_hbm.at[pl.ds(base + i * CHUNK, CHUNK)])

    start_pairs(0, 0)
    start_pairs(1, 1)
    plsc.subcore_barrier()
    wait_pairs(0)
    deint(0)
    start_px(0)

    def body(p, _):
        i = 2 * p
        wait_pairs(1)
        deint(1)
        start_px(1)

        @pl.when(i + 2 < NFULL)
        def _f0():
            start_pairs(i + 2, 0)

        wait_px(0)
        scatter(i, 0)

        @pl.when(i + 2 < NFULL)
        def _g0():
            wait_pairs(0)
            deint(0)
            start_px(0)

        @pl.when(i + 3 < NFULL)
        def _f1():
            start_pairs(i + 3, 1)

        wait_px(1)
        scatter(i + 1, 1)
        return 0

    lax.fori_loop(0, NPAIR, body, 0)

    pltpu.async_copy(
        pairs_hbm.at[pl.ds(2 * base + 2 * CHUNK * NFULL, 2 * TAIL)],
        pb0.at[pl.ds(0, 2 * TAIL)], psems[0]).wait()
    deint(0, groups=TAIL // 16)
    pltpu.async_copy(table_sp.at[ib0.at[pl.ds(0, TAIL)]],
                     r0.at[pl.ds(0, TAIL)], gsems[0]).wait()
    pltpu.sync_copy(r0.at[pl.ds(0, TAIL)],
                    out_hbm.at[pl.ds(base + NFULL * CHUNK, TAIL)])


@jax.jit
def _pix_gather(pairs_flat, px):
    mesh = plsc.VectorSubcoreMesh(core_axis_name="c", subcore_axis_name="s")
    run = functools.partial(
        pl.kernel,
        mesh=mesh,
        out_type=jax.ShapeDtypeStruct((N_EDGES, D_FEAT), jnp.float32),
        scratch_types=[
            pltpu.VMEM_SHARED((N_NODES, D_FEAT), jnp.float32),
            pltpu.VMEM((2 * CHUNK,), jnp.int32),
            pltpu.VMEM((2 * CHUNK,), jnp.int32),
            pltpu.VMEM((CHUNK,), jnp.int32),
            pltpu.VMEM((CHUNK,), jnp.int32),
            pltpu.VMEM((CHUNK, D_FEAT), jnp.float32),
            pltpu.VMEM((CHUNK, D_FEAT), jnp.float32),
            pltpu.SemaphoreType.DMA,
            pltpu.SemaphoreType.DMA,
            pltpu.SemaphoreType.DMA,
            pltpu.SemaphoreType.DMA,
        ],
    )(_gather_kernel)
    return run(pairs_flat, px)


def kernel(ind_2, px):
    return _pix_gather(ind_2.reshape(2 * N_EDGES), px)
